# baseline (speedup 1.0000x reference)
import functools

import numpy as np
import jax
import jax.numpy as jnp
from jax.experimental import pallas as pl

PROB = 0.25


@functools.lru_cache(maxsize=None)
def _drop_indices(b: int):
    cpu = jax.devices("cpu")[0]
    with jax.default_device(cpu):
        perm = np.asarray(jax.random.permutation(jax.random.key(1), b))
    n = 1 if b == 1 else int(b * PROB)
    return tuple(int(i) for i in perm[:n])


def _masked_copy_kernel(x_ref, o_ref, *, drop, t):
    b = pl.program_id(0)
    dropped = functools.reduce(jnp.logical_or, [b == di for di in drop])
    tids = jax.lax.broadcasted_iota(jnp.int32, o_ref.shape, 1)
    x = x_ref[...]
    o_ref[...] = jnp.where(jnp.logical_or(~dropped, tids == (t - 1)), x, 0.0)


@functools.partial(jax.jit, static_argnums=(1,))
def _run(emb0, drop):
    b, t, c, d = emb0.shape
    f = c * d
    x = emb0.reshape(b, t, f)

    out = pl.pallas_call(
        functools.partial(_masked_copy_kernel, drop=drop, t=t),
        grid=(b,),
        in_specs=[pl.BlockSpec((1, t, f), lambda i: (i, 0, 0))],
        out_specs=pl.BlockSpec((1, t, f), lambda i: (i, 0, 0)),
        out_shape=jax.ShapeDtypeStruct((b, t, f), emb0.dtype),
    )(x)
    return out.reshape(b, t, c, d)


_drop_indices(16)


def kernel(emb0):
    return _run(emb0, _drop_indices(emb0.shape[0]))


# speedup vs baseline: 2.9840x; 2.9840x over previous
import functools

import numpy as np
import jax
import jax.numpy as jnp
from jax.experimental import pallas as pl

PROB = 0.25
CH = 512


@functools.lru_cache(maxsize=None)
def _drop_indices(b: int):
    cpu = jax.devices("cpu")[0]
    with jax.default_device(cpu):
        perm = np.asarray(jax.random.permutation(jax.random.key(1), b))
    n = 1 if b == 1 else int(b * PROB)
    return tuple(int(i) for i in perm[:n])


def _is_dropped(i, drop):
    return functools.reduce(jnp.logical_or, [i == di for di in drop])


def _masked_copy_kernel(x_ref, o_ref, *, drop, t, ch):
    i = pl.program_id(0)
    j = pl.program_id(1)
    dropped = _is_dropped(i, drop)

    @pl.when(~dropped)
    def _copy():
        o_ref[...] = x_ref[...]

    @pl.when(dropped)
    def _zero():
        last = x_ref[0, ch - 1, :, :]
        tids = jax.lax.broadcasted_iota(jnp.int32, o_ref.shape, 1) + j * ch
        o_ref[...] = jnp.where(tids == t - 1, last[None, None], 0.0)


@functools.partial(jax.jit, static_argnums=(1,))
def _run(emb0, drop):
    b, t, c, d = emb0.shape
    last_j = t // CH - 1

    def in_map(i, j):
        return (i, jnp.where(_is_dropped(i, drop), last_j, j), 0, 0)

    return pl.pallas_call(
        functools.partial(_masked_copy_kernel, drop=drop, t=t, ch=CH),
        grid=(b, t // CH),
        in_specs=[pl.BlockSpec((1, CH, c, d), in_map)],
        out_specs=pl.BlockSpec((1, CH, c, d), lambda i, j: (i, j, 0, 0)),
        out_shape=jax.ShapeDtypeStruct((b, t, c, d), emb0.dtype),
    )(emb0)


_drop_indices(16)


def kernel(emb0):
    return _run(emb0, _drop_indices(emb0.shape[0]))


# speedup vs baseline: 2.9946x; 1.0036x over previous
import functools

import numpy as np
import jax
import jax.numpy as jnp
from jax.experimental import pallas as pl
from jax.experimental.pallas import tpu as pltpu

PROB = 0.25
CH = 512


@functools.lru_cache(maxsize=None)
def _drop_indices(b: int):
    cpu = jax.devices("cpu")[0]
    with jax.default_device(cpu):
        perm = np.asarray(jax.random.permutation(jax.random.key(1), b))
    n = 1 if b == 1 else int(b * PROB)
    return tuple(int(i) for i in perm[:n])


def _is_dropped(i, drop):
    return functools.reduce(jnp.logical_or, [i == di for di in drop])


def _masked_copy_kernel(x_ref, o_ref, *, drop, t, ch):
    i = pl.program_id(0)
    j = pl.program_id(1)
    dropped = _is_dropped(i, drop)

    @pl.when(~dropped)
    def _copy():
        o_ref[...] = x_ref[...]

    @pl.when(dropped)
    def _zero():
        last = x_ref[0, ch - 1, :, :]
        tids = jax.lax.broadcasted_iota(jnp.int32, o_ref.shape, 1) + j * ch
        o_ref[...] = jnp.where(tids == t - 1, last[None, None], 0.0)


@functools.partial(jax.jit, static_argnums=(1,))
def _run(emb0, drop):
    b, t, c, d = emb0.shape
    last_j = t // CH - 1

    def in_map(i, j):
        return (i, jnp.where(_is_dropped(i, drop), last_j, j), 0, 0)

    return pl.pallas_call(
        functools.partial(_masked_copy_kernel, drop=drop, t=t, ch=CH),
        grid=(b, t // CH),
        in_specs=[pl.BlockSpec((1, CH, c, d), in_map)],
        out_specs=pl.BlockSpec((1, CH, c, d), lambda i, j: (i, j, 0, 0)),
        out_shape=jax.ShapeDtypeStruct((b, t, c, d), emb0.dtype),
        compiler_params=pltpu.CompilerParams(
            dimension_semantics=("parallel", "parallel")),
    )(emb0)


_drop_indices(16)


def kernel(emb0):
    return _run(emb0, _drop_indices(emb0.shape[0]))


# speedup vs baseline: 3.8357x; 1.2809x over previous
import functools

import numpy as np
import jax
import jax.numpy as jnp
from jax.experimental import pallas as pl
from jax.experimental.pallas import tpu as pltpu

PROB = 0.25
CH = 1024


@functools.lru_cache(maxsize=None)
def _drop_indices(b: int):
    cpu = jax.devices("cpu")[0]
    with jax.default_device(cpu):
        perm = np.asarray(jax.random.permutation(jax.random.key(1), b))
    n = 1 if b == 1 else int(b * PROB)
    return tuple(int(i) for i in perm[:n])


def _is_dropped(i, drop):
    return functools.reduce(jnp.logical_or, [i == di for di in drop])


def _masked_copy_kernel(x_ref, o_ref, *, drop, t, ch):
    i = pl.program_id(0)
    j = pl.program_id(1)
    dropped = _is_dropped(i, drop)

    @pl.when(~dropped)
    def _copy():
        o_ref[...] = x_ref[...]

    @pl.when(dropped)
    def _zero():
        last = x_ref[0, ch - 1, :, :]
        tids = jax.lax.broadcasted_iota(jnp.int32, o_ref.shape, 1) + j * ch
        o_ref[...] = jnp.where(tids == t - 1, last[None, None], 0.0)


@functools.partial(jax.jit, static_argnums=(1,))
def _run(emb0, drop):
    b, t, c, d = emb0.shape
    last_j = t // CH - 1

    def in_map(i, j):
        return (i, jnp.where(_is_dropped(i, drop), last_j, j), 0, 0)

    return pl.pallas_call(
        functools.partial(_masked_copy_kernel, drop=drop, t=t, ch=CH),
        grid=(b, t // CH),
        in_specs=[pl.BlockSpec((1, CH, c, d), in_map)],
        out_specs=pl.BlockSpec((1, CH, c, d), lambda i, j: (i, j, 0, 0)),
        out_shape=jax.ShapeDtypeStruct((b, t, c, d), emb0.dtype),
        compiler_params=pltpu.CompilerParams(
            dimension_semantics=("parallel", "parallel")),
    )(emb0)


_drop_indices(16)


def kernel(emb0):
    return _run(emb0, _drop_indices(emb0.shape[0]))


# speedup vs baseline: 4.4061x; 1.1487x over previous
import functools

import numpy as np
import jax
import jax.numpy as jnp
from jax.experimental import pallas as pl
from jax.experimental.pallas import tpu as pltpu

PROB = 0.25
CH = 2048


@functools.lru_cache(maxsize=None)
def _drop_indices(b: int):
    cpu = jax.devices("cpu")[0]
    with jax.default_device(cpu):
        perm = np.asarray(jax.random.permutation(jax.random.key(1), b))
    n = 1 if b == 1 else int(b * PROB)
    return tuple(int(i) for i in perm[:n])


def _is_dropped(i, drop):
    return functools.reduce(jnp.logical_or, [i == di for di in drop])


def _masked_copy_kernel(x_ref, o_ref, *, drop, t, ch):
    i = pl.program_id(0)
    j = pl.program_id(1)
    dropped = _is_dropped(i, drop)

    @pl.when(~dropped)
    def _copy():
        o_ref[...] = x_ref[...]

    @pl.when(dropped)
    def _zero():
        last = x_ref[0, ch - 1, :, :]
        tids = jax.lax.broadcasted_iota(jnp.int32, o_ref.shape, 1) + j * ch
        o_ref[...] = jnp.where(tids == t - 1, last[None, None], 0.0)


@functools.partial(jax.jit, static_argnums=(1,))
def _run(emb0, drop):
    b, t, c, d = emb0.shape
    last_j = t // CH - 1

    def in_map(i, j):
        return (i, jnp.where(_is_dropped(i, drop), last_j, j), 0, 0)

    return pl.pallas_call(
        functools.partial(_masked_copy_kernel, drop=drop, t=t, ch=CH),
        grid=(b, t // CH),
        in_specs=[pl.BlockSpec((1, CH, c, d), in_map)],
        out_specs=pl.BlockSpec((1, CH, c, d), lambda i, j: (i, j, 0, 0)),
        out_shape=jax.ShapeDtypeStruct((b, t, c, d), emb0.dtype),
        compiler_params=pltpu.CompilerParams(
            dimension_semantics=("parallel", "parallel")),
    )(emb0)


_drop_indices(16)


def kernel(emb0):
    return _run(emb0, _drop_indices(emb0.shape[0]))
